# Initial kernel scaffold; baseline (speedup 1.0000x reference)
import jax
import jax.numpy as jnp
from jax.experimental import pallas as pl


def kernel(x, freqs, fluid_params, Wq, Wk, Wv, Wo, ada1_scale, ada1_shift, ada2_scale, ada2_shift, Wr, router_bias, W1, b1, W2, b2):
    raise NotImplementedError("write your pallas kernel here")


# speedup vs baseline: 1.2563x; 1.2563x over previous
import functools

import jax
import jax.numpy as jnp
import numpy as np
from jax.experimental import pallas as pl
from jax.experimental.pallas import tpu as pltpu

B, S, D, H, F = 1, 2048, 1024, 16, 8
E, K, I = 8, 2, 4096
HD = D // H
HH = HD // 2
CAP = 768
SBLK = 256
NSB = S // SBLK
IBLK = 1024
NIB = I // IBLK

_f32 = jnp.float32
_bf16 = jnp.bfloat16


def _dims(*sems):
    return pltpu.CompilerParams(dimension_semantics=sems)


def _qkv_body(x_ref, fp_ref, ws_ref, wb_ref, fr_ref,
              wqa_ref, wqb_ref, wka_ref, wkb_ref, wv_ref,
              q_ref, k_ref, v_ref):
    x = x_ref[...]
    mu = jnp.mean(x, axis=1, keepdims=True)
    var = jnp.mean((x - mu) ** 2, axis=1, keepdims=True)
    xn = (x - mu) / jnp.sqrt(var + 1e-6)
    fp = fp_ref[...]
    scale = jax.lax.dot_general(fp, ws_ref[...], (((1,), (0,)), ((), ())),
                                preferred_element_type=_f32)
    shift = jax.lax.dot_general(fp, wb_ref[...], (((1,), (0,)), ((), ())),
                                preferred_element_type=_f32)
    h = xn * (1.0 + scale) + shift

    def mm(w_ref):
        return jax.lax.dot_general(h, w_ref[...], (((1,), (0,)), ((), ())),
                                   preferred_element_type=_f32)

    q1, q2 = mm(wqa_ref), mm(wqb_ref)
    k1, k2 = mm(wka_ref), mm(wkb_ref)
    v_ref[...] = mm(wv_ref)

    fr = fr_ref[...]
    cos = jnp.cos(fr)
    sin = jnp.sin(fr)
    cosT = jnp.concatenate([cos] * H, axis=1)
    sinT = jnp.concatenate([sin] * H, axis=1)
    q_ref[...] = jnp.concatenate(
        [q1 * cosT - q2 * sinT, q1 * sinT + q2 * cosT], axis=1)
    k_ref[...] = jnp.concatenate(
        [k1 * cosT - k2 * sinT, k1 * sinT + k2 * cosT], axis=1)


def _qkv_call(x2, fp, ws, wb, freqs, wqa, wqb, wka, wkb, wv):
    return pl.pallas_call(
        _qkv_body,
        grid=(NSB,),
        in_specs=[
            pl.BlockSpec((SBLK, D), lambda i: (i, 0)),
            pl.BlockSpec((1, F), lambda i: (0, 0)),
            pl.BlockSpec((F, D), lambda i: (0, 0)),
            pl.BlockSpec((F, D), lambda i: (0, 0)),
            pl.BlockSpec((SBLK, HH), lambda i: (i, 0)),
            pl.BlockSpec((D, D // 2), lambda i: (0, 0)),
            pl.BlockSpec((D, D // 2), lambda i: (0, 0)),
            pl.BlockSpec((D, D // 2), lambda i: (0, 0)),
            pl.BlockSpec((D, D // 2), lambda i: (0, 0)),
            pl.BlockSpec((D, D), lambda i: (0, 0)),
        ],
        out_specs=[
            pl.BlockSpec((SBLK, D), lambda i: (i, 0)),
            pl.BlockSpec((SBLK, D), lambda i: (i, 0)),
            pl.BlockSpec((SBLK, D), lambda i: (i, 0)),
        ],
        out_shape=[jax.ShapeDtypeStruct((S, D), _f32)] * 3,
        compiler_params=_dims("arbitrary"),
    )(x2, fp, ws, wb, freqs, wqa, wqb, wka, wkb, wv)


def _attn_body(q_ref, k_ref, v_ref, o_ref):
    q = q_ref[0]
    k = k_ref[0]
    v = v_ref[0]
    s = jax.lax.dot_general(q, k, (((1,), (1,)), ((), ())),
                            preferred_element_type=_f32) * (1.0 / np.sqrt(HD))
    m = jnp.max(s, axis=1, keepdims=True)
    p = jnp.exp(s - m)
    l = jnp.sum(p, axis=1, keepdims=True)
    a = p / l
    o_ref[0] = jax.lax.dot_general(a, v, (((1,), (0,)), ((), ())),
                                   preferred_element_type=_f32)


def _attn_call(q3, k3, v3):
    return pl.pallas_call(
        _attn_body,
        grid=(H, NSB),
        in_specs=[
            pl.BlockSpec((1, SBLK, HD), lambda h, i: (h, i, 0)),
            pl.BlockSpec((1, S, HD), lambda h, i: (h, 0, 0)),
            pl.BlockSpec((1, S, HD), lambda h, i: (h, 0, 0)),
        ],
        out_specs=pl.BlockSpec((1, SBLK, HD), lambda h, i: (h, i, 0)),
        out_shape=jax.ShapeDtypeStruct((H, S, HD), _f32),
        compiler_params=_dims("arbitrary", "arbitrary"),
    )(q3, k3, v3)


def _post_body(o_ref, x_ref, wo_ref, fp_ref, ws_ref, wb_ref, wr_ref,
               x1_ref, h2_ref, sc_ref):
    o = o_ref[...]
    x1 = x_ref[...] + jax.lax.dot_general(
        o, wo_ref[...], (((1,), (0,)), ((), ())), preferred_element_type=_f32)
    x1_ref[...] = x1
    mu = jnp.mean(x1, axis=1, keepdims=True)
    var = jnp.mean((x1 - mu) ** 2, axis=1, keepdims=True)
    xn = (x1 - mu) / jnp.sqrt(var + 1e-6)
    fp = fp_ref[...]
    scale = jax.lax.dot_general(fp, ws_ref[...], (((1,), (0,)), ((), ())),
                                preferred_element_type=_f32)
    shift = jax.lax.dot_general(fp, wb_ref[...], (((1,), (0,)), ((), ())),
                                preferred_element_type=_f32)
    h2 = xn * (1.0 + scale) + shift
    h2_ref[...] = h2
    sc_ref[...] = jax.lax.dot_general(h2, wr_ref[...], (((1,), (0,)), ((), ())),
                                      preferred_element_type=_f32)


def _post_call(o2, x2, wo, fp, ws, wb, wr_pad):
    return pl.pallas_call(
        _post_body,
        grid=(NSB,),
        in_specs=[
            pl.BlockSpec((SBLK, D), lambda i: (i, 0)),
            pl.BlockSpec((SBLK, D), lambda i: (i, 0)),
            pl.BlockSpec((D, D), lambda i: (0, 0)),
            pl.BlockSpec((1, F), lambda i: (0, 0)),
            pl.BlockSpec((F, D), lambda i: (0, 0)),
            pl.BlockSpec((F, D), lambda i: (0, 0)),
            pl.BlockSpec((D, 128), lambda i: (0, 0)),
        ],
        out_specs=[
            pl.BlockSpec((SBLK, D), lambda i: (i, 0)),
            pl.BlockSpec((SBLK, D), lambda i: (i, 0)),
            pl.BlockSpec((SBLK, 128), lambda i: (i, 0)),
        ],
        out_shape=[
            jax.ShapeDtypeStruct((S, D), _f32),
            jax.ShapeDtypeStruct((S, D), _f32),
            jax.ShapeDtypeStruct((S, 128), _f32),
        ],
        compiler_params=_dims("arbitrary"),
    )(o2, x2, wo, fp, ws, wb, wr_pad)


def _router_body(sc_ref, rb_ref, meta_ref, idx_ref):
    sc = sc_ref[...]
    lane = jax.lax.broadcasted_iota(jnp.int32, (S, 128), 1)
    valid = lane < E
    neg = jnp.float32(-1e30)
    biased = jnp.where(valid, sc + rb_ref[...], neg)
    m1 = jnp.max(biased, axis=1, keepdims=True)
    e1 = jnp.min(jnp.where((biased == m1) & valid, lane, 999), axis=1,
                 keepdims=True)
    b2 = jnp.where(lane == e1, neg, biased)
    m2 = jnp.max(b2, axis=1, keepdims=True)
    e2 = jnp.min(jnp.where((b2 == m2) & valid, lane, 999), axis=1,
                 keepdims=True)
    s1 = jnp.sum(jnp.where(lane == e1, sc, 0.0), axis=1, keepdims=True)
    s2 = jnp.sum(jnp.where(lane == e2, sc, 0.0), axis=1, keepdims=True)
    mx = jnp.maximum(s1, s2)
    p1 = jnp.exp(s1 - mx)
    p2 = jnp.exp(s2 - mx)
    g1 = p1 / (p1 + p2)
    g2 = p2 / (p1 + p2)

    mask = ((lane == e1) | (lane == e2)).astype(_f32)
    row = jax.lax.broadcasted_iota(jnp.int32, (S, S), 0)
    col = jax.lax.broadcasted_iota(jnp.int32, (S, S), 1)
    ltri = (row > col).astype(_f32)
    pos = jax.lax.dot_general(ltri, mask, (((1,), (0,)), ((), ())),
                              preferred_element_type=_f32)
    pos0 = jnp.sum(jnp.where(lane == e1, pos, 0.0), axis=1, keepdims=True)
    pos1 = jnp.sum(jnp.where(lane == e2, pos, 0.0), axis=1, keepdims=True)

    meta_ref[...] = jnp.concatenate(
        [g1, g2, pos0, pos1] + [jnp.zeros((S, 124), _f32)], axis=1)
    idx_ref[...] = jnp.concatenate(
        [e1, e2] + [jnp.zeros((S, 126), jnp.int32)], axis=1)


def _router_call(scores, rb_pad):
    return pl.pallas_call(
        _router_body,
        grid=(1,),
        in_specs=[
            pl.BlockSpec((S, 128), lambda i: (0, 0)),
            pl.BlockSpec((1, 128), lambda i: (0, 0)),
        ],
        out_specs=[
            pl.BlockSpec((S, 128), lambda i: (0, 0)),
            pl.BlockSpec((S, 128), lambda i: (0, 0)),
        ],
        out_shape=[
            jax.ShapeDtypeStruct((S, 128), _f32),
            jax.ShapeDtypeStruct((S, 128), jnp.int32),
        ],
        compiler_params=_dims("arbitrary"),
    )(scores, rb_pad)


def _dispatch_body(idx_ref, meta_ref, h2_ref, buf_ref):
    e = pl.program_id(0)
    e1 = idx_ref[:, 0:1]
    e2 = idx_ref[:, 1:2]
    pos0 = meta_ref[:, 2:3]
    pos1 = meta_ref[:, 3:4]
    pp = jax.lax.broadcasted_iota(jnp.int32, (S, CAP), 1).astype(_f32)
    c0 = (e1 == e) & (pos0 == pp) & (pos0 < CAP)
    c1 = (e2 == e) & (pos1 == pp) & (pos1 < CAP)
    oh = (c0 | c1).astype(_bf16)
    h2 = h2_ref[...].astype(_bf16)
    buf_ref[0] = jax.lax.dot_general(
        oh, h2, (((0,), (0,)), ((), ())),
        preferred_element_type=_f32).astype(_bf16)


def _dispatch_call(idx_p, meta, h2):
    return pl.pallas_call(
        _dispatch_body,
        grid=(E,),
        in_specs=[
            pl.BlockSpec((S, 128), lambda e: (0, 0)),
            pl.BlockSpec((S, 128), lambda e: (0, 0)),
            pl.BlockSpec((S, D), lambda e: (0, 0)),
        ],
        out_specs=pl.BlockSpec((1, CAP, D), lambda e: (e, 0, 0)),
        out_shape=jax.ShapeDtypeStruct((E, CAP, D), _bf16),
        compiler_params=_dims("arbitrary"),
    )(idx_p, meta, h2)


def _ffn_body(buf_ref, w1_ref, b1_ref, w2_ref, b2_ref, y_ref):
    i = pl.program_id(1)
    bufb = buf_ref[0]
    w1 = w1_ref[0].astype(_bf16)
    h = jax.lax.dot_general(bufb, w1, (((1,), (0,)), ((), ())),
                            preferred_element_type=_f32) + b1_ref[0]
    g = jax.nn.gelu(h).astype(_bf16)
    w2 = w2_ref[0].astype(_bf16)
    part = jax.lax.dot_general(g, w2, (((1,), (0,)), ((), ())),
                               preferred_element_type=_f32)

    @pl.when(i == 0)
    def _():
        y_ref[0] = part + b2_ref[0]

    @pl.when(i > 0)
    def _():
        y_ref[0] += part


def _ffn_call(buf, w1, b1, w2, b2):
    return pl.pallas_call(
        _ffn_body,
        grid=(E, NIB),
        in_specs=[
            pl.BlockSpec((1, CAP, D), lambda e, i: (e, 0, 0)),
            pl.BlockSpec((1, D, IBLK), lambda e, i: (e, 0, i)),
            pl.BlockSpec((1, 1, IBLK), lambda e, i: (e, 0, i)),
            pl.BlockSpec((1, IBLK, D), lambda e, i: (e, i, 0)),
            pl.BlockSpec((1, 1, D), lambda e, i: (e, 0, 0)),
        ],
        out_specs=pl.BlockSpec((1, CAP, D), lambda e, i: (e, 0, 0)),
        out_shape=jax.ShapeDtypeStruct((E, CAP, D), _f32),
        compiler_params=_dims("arbitrary", "arbitrary"),
    )(buf, w1, b1, w2, b2)


def _combine_body(idx_ref, meta_ref, y_ref, x1_ref, out_ref):
    e = pl.program_id(0)
    e1 = idx_ref[:, 0:1]
    e2 = idx_ref[:, 1:2]
    g1 = meta_ref[:, 0:1]
    g2 = meta_ref[:, 1:2]
    pos0 = meta_ref[:, 2:3]
    pos1 = meta_ref[:, 3:4]
    pp = jax.lax.broadcasted_iota(jnp.int32, (S, CAP), 1).astype(_f32)
    c0 = (e1 == e) & (pos0 == pp) & (pos0 < CAP)
    c1 = (e2 == e) & (pos1 == pp) & (pos1 < CAP)
    cmat = (jnp.where(c0, g1, 0.0) + jnp.where(c1, g2, 0.0)).astype(_bf16)
    yb = y_ref[0].astype(_bf16)
    add = jax.lax.dot_general(cmat, yb, (((1,), (0,)), ((), ())),
                              preferred_element_type=_f32)

    @pl.when(e == 0)
    def _():
        out_ref[...] = x1_ref[...] + add

    @pl.when(e > 0)
    def _():
        out_ref[...] += add


def _combine_call(idx_p, meta, y, x1):
    return pl.pallas_call(
        _combine_body,
        grid=(E,),
        in_specs=[
            pl.BlockSpec((S, 128), lambda e: (0, 0)),
            pl.BlockSpec((S, 128), lambda e: (0, 0)),
            pl.BlockSpec((1, CAP, D), lambda e: (e, 0, 0)),
            pl.BlockSpec((S, D), lambda e: (0, 0)),
        ],
        out_specs=pl.BlockSpec((S, D), lambda e: (0, 0)),
        out_shape=jax.ShapeDtypeStruct((S, D), _f32),
        compiler_params=_dims("arbitrary"),
    )(idx_p, meta, y, x1)


def kernel(x, freqs, fluid_params, Wq, Wk, Wv, Wo, ada1_scale, ada1_shift,
           ada2_scale, ada2_shift, Wr, router_bias, W1, b1, W2, b2):
    x2 = x.reshape(S, D)

    def split(w):
        wr = w.reshape(D, H, HH, 2)
        return (wr[..., 0].reshape(D, H * HH), wr[..., 1].reshape(D, H * HH))

    wqa, wqb = split(Wq)
    wka, wkb = split(Wk)

    q, k, v = _qkv_call(x2, fluid_params, ada1_scale, ada1_shift, freqs,
                        wqa, wqb, wka, wkb, Wv)

    q3 = q.reshape(S, 2, H, HH).transpose(2, 0, 1, 3).reshape(H, S, HD)
    k3 = k.reshape(S, 2, H, HH).transpose(2, 0, 1, 3).reshape(H, S, HD)
    v3 = v.reshape(S, H, HD).transpose(1, 0, 2)

    o3 = _attn_call(q3, k3, v3)
    o2 = o3.transpose(1, 0, 2).reshape(S, D)

    wr_pad = jnp.pad(Wr, ((0, 0), (0, 128 - E)))
    x1, h2, scores = _post_call(o2, x2, Wo, fluid_params,
                                ada2_scale, ada2_shift, wr_pad)

    rb_pad = jnp.pad(router_bias.reshape(1, E), ((0, 0), (0, 128 - E)))
    meta, idx_p = _router_call(scores, rb_pad)

    buf = _dispatch_call(idx_p, meta, h2)
    y = _ffn_call(buf, W1, b1.reshape(E, 1, I), W2, b2.reshape(E, 1, D))
    out = _combine_call(idx_p, meta, y, x1)

    gates = meta[:, :K]
    idx = idx_p[:, :K]
    return (out.reshape(B, S, D), gates, idx)


# speedup vs baseline: 1.4204x; 1.1306x over previous
import functools

import jax
import jax.numpy as jnp
import numpy as np
from jax.experimental import pallas as pl
from jax.experimental.pallas import tpu as pltpu

B, S, D, H, F = 1, 2048, 1024, 16, 8
E, K, I = 8, 2, 4096
HD = D // H
HH = HD // 2
CAP = 768
SBLK = 256
NSB = S // SBLK
IBLK = 1024
NIB = I // IBLK

_f32 = jnp.float32
_bf16 = jnp.bfloat16


def _dims(*sems):
    return pltpu.CompilerParams(dimension_semantics=sems)


def _qkv_body(x_ref, fp_ref, ws_ref, wb_ref, fr_ref,
              wqa_ref, wqb_ref, wka_ref, wkb_ref, wv_ref,
              q_ref, k_ref, v_ref):
    x = x_ref[...]
    mu = jnp.mean(x, axis=1, keepdims=True)
    var = jnp.mean((x - mu) ** 2, axis=1, keepdims=True)
    xn = (x - mu) / jnp.sqrt(var + 1e-6)
    fp = fp_ref[...]
    scale = jax.lax.dot_general(fp, ws_ref[...], (((1,), (0,)), ((), ())),
                                preferred_element_type=_f32)
    shift = jax.lax.dot_general(fp, wb_ref[...], (((1,), (0,)), ((), ())),
                                preferred_element_type=_f32)
    h = xn * (1.0 + scale) + shift

    def mm(w_ref):
        return jax.lax.dot_general(h, w_ref[...], (((1,), (0,)), ((), ())),
                                   preferred_element_type=_f32)

    q1, q2 = mm(wqa_ref), mm(wqb_ref)
    k1, k2 = mm(wka_ref), mm(wkb_ref)
    v_ref[...] = mm(wv_ref)

    fr = fr_ref[...]
    cos = jnp.cos(fr)
    sin = jnp.sin(fr)
    cosT = jnp.concatenate([cos] * H, axis=1)
    sinT = jnp.concatenate([sin] * H, axis=1)
    q_ref[...] = jnp.concatenate(
        [q1 * cosT - q2 * sinT, q1 * sinT + q2 * cosT], axis=1)
    k_ref[...] = jnp.concatenate(
        [k1 * cosT - k2 * sinT, k1 * sinT + k2 * cosT], axis=1)


def _qkv_call(x2, fp, ws, wb, freqs, wqa, wqb, wka, wkb, wv):
    return pl.pallas_call(
        _qkv_body,
        grid=(NSB,),
        in_specs=[
            pl.BlockSpec((SBLK, D), lambda i: (i, 0)),
            pl.BlockSpec((1, F), lambda i: (0, 0)),
            pl.BlockSpec((F, D), lambda i: (0, 0)),
            pl.BlockSpec((F, D), lambda i: (0, 0)),
            pl.BlockSpec((SBLK, HH), lambda i: (i, 0)),
            pl.BlockSpec((D, D // 2), lambda i: (0, 0)),
            pl.BlockSpec((D, D // 2), lambda i: (0, 0)),
            pl.BlockSpec((D, D // 2), lambda i: (0, 0)),
            pl.BlockSpec((D, D // 2), lambda i: (0, 0)),
            pl.BlockSpec((D, D), lambda i: (0, 0)),
        ],
        out_specs=[
            pl.BlockSpec((SBLK, D), lambda i: (i, 0)),
            pl.BlockSpec((SBLK, D), lambda i: (i, 0)),
            pl.BlockSpec((SBLK, D), lambda i: (i, 0)),
        ],
        out_shape=[jax.ShapeDtypeStruct((S, D), _f32)] * 3,
        compiler_params=_dims("arbitrary"),
    )(x2, fp, ws, wb, freqs, wqa, wqb, wka, wkb, wv)


def _attn_body(q_ref, k_ref, v_ref, o_ref):
    q = q_ref[0]
    k = k_ref[0]
    v = v_ref[0]
    s = jax.lax.dot_general(q, k, (((1,), (1,)), ((), ())),
                            preferred_element_type=_f32) * (1.0 / np.sqrt(HD))
    m = jnp.max(s, axis=1, keepdims=True)
    p = jnp.exp(s - m)
    l = jnp.sum(p, axis=1, keepdims=True)
    o = jax.lax.dot_general(p, v, (((1,), (0,)), ((), ())),
                            preferred_element_type=_f32)
    o_ref[0] = o / l


QBLK = 512
NQB = S // QBLK


def _attn_call(q3, k3, v3):
    return pl.pallas_call(
        _attn_body,
        grid=(H, NQB),
        in_specs=[
            pl.BlockSpec((1, QBLK, HD), lambda h, i: (h, i, 0)),
            pl.BlockSpec((1, S, HD), lambda h, i: (h, 0, 0)),
            pl.BlockSpec((1, S, HD), lambda h, i: (h, 0, 0)),
        ],
        out_specs=pl.BlockSpec((1, QBLK, HD), lambda h, i: (h, i, 0)),
        out_shape=jax.ShapeDtypeStruct((H, S, HD), _f32),
        compiler_params=_dims("arbitrary", "arbitrary"),
    )(q3, k3, v3)


def _post_body(o_ref, x_ref, wo_ref, fp_ref, ws_ref, wb_ref, wr_ref,
               x1_ref, h2_ref, sc_ref):
    o = o_ref[...]
    x1 = x_ref[...] + jax.lax.dot_general(
        o, wo_ref[...], (((1,), (0,)), ((), ())), preferred_element_type=_f32)
    x1_ref[...] = x1
    mu = jnp.mean(x1, axis=1, keepdims=True)
    var = jnp.mean((x1 - mu) ** 2, axis=1, keepdims=True)
    xn = (x1 - mu) / jnp.sqrt(var + 1e-6)
    fp = fp_ref[...]
    scale = jax.lax.dot_general(fp, ws_ref[...], (((1,), (0,)), ((), ())),
                                preferred_element_type=_f32)
    shift = jax.lax.dot_general(fp, wb_ref[...], (((1,), (0,)), ((), ())),
                                preferred_element_type=_f32)
    h2 = xn * (1.0 + scale) + shift
    h2_ref[...] = h2.astype(_bf16)
    sc_ref[...] = jax.lax.dot_general(h2, wr_ref[...], (((1,), (0,)), ((), ())),
                                      preferred_element_type=_f32)


def _post_call(o2, x2, wo, fp, ws, wb, wr_pad):
    return pl.pallas_call(
        _post_body,
        grid=(NSB,),
        in_specs=[
            pl.BlockSpec((SBLK, D), lambda i: (i, 0)),
            pl.BlockSpec((SBLK, D), lambda i: (i, 0)),
            pl.BlockSpec((D, D), lambda i: (0, 0)),
            pl.BlockSpec((1, F), lambda i: (0, 0)),
            pl.BlockSpec((F, D), lambda i: (0, 0)),
            pl.BlockSpec((F, D), lambda i: (0, 0)),
            pl.BlockSpec((D, 128), lambda i: (0, 0)),
        ],
        out_specs=[
            pl.BlockSpec((SBLK, D), lambda i: (i, 0)),
            pl.BlockSpec((SBLK, D), lambda i: (i, 0)),
            pl.BlockSpec((SBLK, 128), lambda i: (i, 0)),
        ],
        out_shape=[
            jax.ShapeDtypeStruct((S, D), _f32),
            jax.ShapeDtypeStruct((S, D), _bf16),
            jax.ShapeDtypeStruct((S, 128), _f32),
        ],
        compiler_params=_dims("arbitrary"),
    )(o2, x2, wo, fp, ws, wb, wr_pad)


def _router_body(sc_ref, rb_ref, meta_ref, idx_ref):
    sc = sc_ref[...]
    lane = jax.lax.broadcasted_iota(jnp.int32, (S, 128), 1)
    valid = lane < E
    neg = jnp.float32(-1e30)
    biased = jnp.where(valid, sc + rb_ref[...], neg)
    m1 = jnp.max(biased, axis=1, keepdims=True)
    e1 = jnp.min(jnp.where((biased == m1) & valid, lane, 999), axis=1,
                 keepdims=True)
    b2 = jnp.where(lane == e1, neg, biased)
    m2 = jnp.max(b2, axis=1, keepdims=True)
    e2 = jnp.min(jnp.where((b2 == m2) & valid, lane, 999), axis=1,
                 keepdims=True)
    s1 = jnp.sum(jnp.where(lane == e1, sc, 0.0), axis=1, keepdims=True)
    s2 = jnp.sum(jnp.where(lane == e2, sc, 0.0), axis=1, keepdims=True)
    mx = jnp.maximum(s1, s2)
    p1 = jnp.exp(s1 - mx)
    p2 = jnp.exp(s2 - mx)
    g1 = p1 / (p1 + p2)
    g2 = p2 / (p1 + p2)

    mask = ((lane == e1) | (lane == e2)).astype(_f32)
    row = jax.lax.broadcasted_iota(jnp.int32, (S, S), 0)
    col = jax.lax.broadcasted_iota(jnp.int32, (S, S), 1)
    ltri = (row > col).astype(_f32)
    pos = jax.lax.dot_general(ltri, mask, (((1,), (0,)), ((), ())),
                              preferred_element_type=_f32)
    pos0 = jnp.sum(jnp.where(lane == e1, pos, 0.0), axis=1, keepdims=True)
    pos1 = jnp.sum(jnp.where(lane == e2, pos, 0.0), axis=1, keepdims=True)

    meta_ref[...] = jnp.concatenate(
        [g1, g2, pos0, pos1] + [jnp.zeros((S, 124), _f32)], axis=1)
    idx_ref[...] = jnp.concatenate(
        [e1, e2] + [jnp.zeros((S, 126), jnp.int32)], axis=1)


def _router_call(scores, rb_pad):
    return pl.pallas_call(
        _router_body,
        grid=(1,),
        in_specs=[
            pl.BlockSpec((S, 128), lambda i: (0, 0)),
            pl.BlockSpec((1, 128), lambda i: (0, 0)),
        ],
        out_specs=[
            pl.BlockSpec((S, 128), lambda i: (0, 0)),
            pl.BlockSpec((S, 128), lambda i: (0, 0)),
        ],
        out_shape=[
            jax.ShapeDtypeStruct((S, 128), _f32),
            jax.ShapeDtypeStruct((S, 128), jnp.int32),
        ],
        compiler_params=_dims("arbitrary"),
    )(scores, rb_pad)


def _dispatch_body(idx_ref, meta_ref, h2_ref, buf_ref):
    e = pl.program_id(0)
    e1 = idx_ref[:, 0:1]
    e2 = idx_ref[:, 1:2]
    pos0 = meta_ref[:, 2:3]
    pos1 = meta_ref[:, 3:4]
    a0 = jnp.where(e1 == e, pos0, -1.0)
    a1 = jnp.where(e2 == e, pos1, -1.0)
    pp = jax.lax.broadcasted_iota(jnp.int32, (S, CAP), 1).astype(_f32)
    oh = ((a0 == pp) | (a1 == pp)).astype(_bf16)
    h2 = h2_ref[...]
    buf_ref[0] = jax.lax.dot_general(
        oh, h2, (((0,), (0,)), ((), ())),
        preferred_element_type=_f32).astype(_bf16)


def _dispatch_call(idx_p, meta, h2):
    return pl.pallas_call(
        _dispatch_body,
        grid=(E,),
        in_specs=[
            pl.BlockSpec((S, 128), lambda e: (0, 0)),
            pl.BlockSpec((S, 128), lambda e: (0, 0)),
            pl.BlockSpec((S, D), lambda e: (0, 0)),
        ],
        out_specs=pl.BlockSpec((1, CAP, D), lambda e: (e, 0, 0)),
        out_shape=jax.ShapeDtypeStruct((E, CAP, D), _bf16),
        compiler_params=_dims("arbitrary"),
    )(idx_p, meta, h2)


def _ffn_body(buf_ref, w1_ref, b1_ref, w2_ref, b2_ref, y_ref):
    i = pl.program_id(1)
    bufb = buf_ref[0]
    w1 = w1_ref[0].astype(_bf16)
    h = jax.lax.dot_general(bufb, w1, (((1,), (0,)), ((), ())),
                            preferred_element_type=_f32) + b1_ref[0]
    g = jax.nn.gelu(h).astype(_bf16)
    w2 = w2_ref[0].astype(_bf16)
    part = jax.lax.dot_general(g, w2, (((1,), (0,)), ((), ())),
                               preferred_element_type=_f32)

    @pl.when(i == 0)
    def _():
        y_ref[0] = part + b2_ref[0]

    @pl.when(i > 0)
    def _():
        y_ref[0] += part


def _ffn_call(buf, w1, b1, w2, b2):
    return pl.pallas_call(
        _ffn_body,
        grid=(E, NIB),
        in_specs=[
            pl.BlockSpec((1, CAP, D), lambda e, i: (e, 0, 0)),
            pl.BlockSpec((1, D, IBLK), lambda e, i: (e, 0, i)),
            pl.BlockSpec((1, 1, IBLK), lambda e, i: (e, 0, i)),
            pl.BlockSpec((1, IBLK, D), lambda e, i: (e, i, 0)),
            pl.BlockSpec((1, 1, D), lambda e, i: (e, 0, 0)),
        ],
        out_specs=pl.BlockSpec((1, CAP, D), lambda e, i: (e, 0, 0)),
        out_shape=jax.ShapeDtypeStruct((E, CAP, D), _f32),
        compiler_params=_dims("arbitrary", "arbitrary"),
    )(buf, w1, b1, w2, b2)


def _combine_body(idx_ref, meta_ref, y_ref, x1_ref, out_ref):
    e = pl.program_id(0)
    e1 = idx_ref[:, 0:1]
    e2 = idx_ref[:, 1:2]
    g1 = meta_ref[:, 0:1]
    g2 = meta_ref[:, 1:2]
    pos0 = meta_ref[:, 2:3]
    pos1 = meta_ref[:, 3:4]
    a0 = jnp.where(e1 == e, pos0, -1.0)
    a1 = jnp.where(e2 == e, pos1, -1.0)
    pp = jax.lax.broadcasted_iota(jnp.int32, (S, CAP), 1).astype(_f32)
    cmat = (jnp.where(a0 == pp, g1, 0.0) +
            jnp.where(a1 == pp, g2, 0.0)).astype(_bf16)
    yb = y_ref[0].astype(_bf16)
    add = jax.lax.dot_general(cmat, yb, (((1,), (0,)), ((), ())),
                              preferred_element_type=_f32)

    @pl.when(e == 0)
    def _():
        out_ref[...] = x1_ref[...] + add

    @pl.when(e > 0)
    def _():
        out_ref[...] += add


def _combine_call(idx_p, meta, y, x1):
    return pl.pallas_call(
        _combine_body,
        grid=(E,),
        in_specs=[
            pl.BlockSpec((S, 128), lambda e: (0, 0)),
            pl.BlockSpec((S, 128), lambda e: (0, 0)),
            pl.BlockSpec((1, CAP, D), lambda e: (e, 0, 0)),
            pl.BlockSpec((S, D), lambda e: (0, 0)),
        ],
        out_specs=pl.BlockSpec((S, D), lambda e: (0, 0)),
        out_shape=jax.ShapeDtypeStruct((S, D), _f32),
        compiler_params=_dims("arbitrary"),
    )(idx_p, meta, y, x1)


def kernel(x, freqs, fluid_params, Wq, Wk, Wv, Wo, ada1_scale, ada1_shift,
           ada2_scale, ada2_shift, Wr, router_bias, W1, b1, W2, b2):
    x2 = x.reshape(S, D)

    def split(w):
        wr = w.reshape(D, H, HH, 2)
        return (wr[..., 0].reshape(D, H * HH), wr[..., 1].reshape(D, H * HH))

    wqa, wqb = split(Wq)
    wka, wkb = split(Wk)

    q, k, v = _qkv_call(x2, fluid_params, ada1_scale, ada1_shift, freqs,
                        wqa, wqb, wka, wkb, Wv)

    q3 = q.reshape(S, 2, H, HH).transpose(2, 0, 1, 3).reshape(H, S, HD)
    k3 = k.reshape(S, 2, H, HH).transpose(2, 0, 1, 3).reshape(H, S, HD)
    v3 = v.reshape(S, H, HD).transpose(1, 0, 2)

    o3 = _attn_call(q3, k3, v3)
    o2 = o3.transpose(1, 0, 2).reshape(S, D)

    wr_pad = jnp.pad(Wr, ((0, 0), (0, 128 - E)))
    x1, h2, scores = _post_call(o2, x2, Wo, fluid_params,
                                ada2_scale, ada2_shift, wr_pad)

    rb_pad = jnp.pad(router_bias.reshape(1, E), ((0, 0), (0, 128 - E)))
    meta, idx_p = _router_call(scores, rb_pad)

    buf = _dispatch_call(idx_p, meta, h2)
    y = _ffn_call(buf, W1, b1.reshape(E, 1, I), W2, b2.reshape(E, 1, D))
    out = _combine_call(idx_p, meta, y, x1)

    gates = meta[:, :K]
    idx = idx_p[:, :K]
    return (out.reshape(B, S, D), gates, idx)
